# baseline (speedup 1.0000x reference)
import functools

import jax
import jax.numpy as jnp
from jax import lax
from jax.experimental import pallas as pl
from jax.experimental.pallas import tpu as pltpu
from jax.experimental.pallas import tpu_sc as plsc

_PCHUNK = 8
_DEPTH = 3


def _emb_call(ids_flat, word_emb, pos_emb, n_batch, seq_len):
    n_rows, = ids_flat.shape
    _, d = word_emb.shape

    info = plsc.get_sparse_core_info()
    nc, ns, lanes = info.num_cores, info.num_subcores, info.num_lanes
    nw = nc * ns
    b_per_w = n_rows // nw
    s_per_w = seq_len // nw
    pchunk = _PCHUNK
    npc = s_per_w // pchunk
    depth = _DEPTH

    mesh = plsc.VectorSubcoreMesh(core_axis_name="c", subcore_axis_name="s")

    scratch = [
        pltpu.VMEM((b_per_w,), jnp.int32),
        pltpu.VMEM((depth * pchunk, d), jnp.float32),
        pltpu.VMEM((depth * n_batch * pchunk, d), jnp.float32),
    ]
    scratch += [pltpu.SemaphoreType.DMA for _ in range(3 * depth + 1)]

    @functools.partial(
        pl.kernel,
        mesh=mesh,
        out_type=jax.ShapeDtypeStruct((n_rows, d), jnp.float32),
        scratch_types=scratch,
    )
    def emb_kernel(ids_hbm, wemb_hbm, pemb_hbm, out_hbm, idx_v, pos_a, rows_a,
                   *sems):
        psem = sems[:depth]
        gsem = sems[depth:2 * depth]
        osem = sems[2 * depth:3 * depth]
        isem = sems[3 * depth]

        wid = lax.axis_index("s") * nc + lax.axis_index("c")
        pos0 = wid * s_per_w

        ih = [
            pltpu.async_copy(
                ids_hbm.at[pl.ds(bb * seq_len + pos0, s_per_w)],
                idx_v.at[pl.ds(bb * s_per_w, s_per_w)],
                isem,
            )
            for bb in range(n_batch)
        ]
        for h in ih:
            h.wait()

        ph = [None] * npc
        gh = [None] * npc
        oh = [None] * npc

        def start(p):
            s = p % depth
            roff = p * pchunk
            ph[p] = pltpu.async_copy(
                pemb_hbm.at[pl.ds(pos0 + roff, pchunk)],
                pos_a.at[pl.ds(s * pchunk, pchunk)],
                psem[s],
            )
            gh[p] = [
                pltpu.async_copy(
                    wemb_hbm.at[idx_v.at[pl.ds(bb * s_per_w + roff, pchunk)]],
                    rows_a.at[pl.ds((s * n_batch + bb) * pchunk, pchunk)],
                    gsem[s],
                )
                for bb in range(n_batch)
            ]

        def process(p):
            s = p % depth
            roff = p * pchunk
            ph[p].wait()
            for h in gh[p]:
                h.wait()

            def body(r, carry):
                for j in range(d // lanes):
                    sl = pl.ds(j * lanes, lanes)
                    pv = pos_a[s * pchunk + r, sl]
                    for bb in range(n_batch):
                        plsc.addupdate(
                            rows_a.at[(s * n_batch + bb) * pchunk + r, sl], pv
                        )
                return carry

            lax.fori_loop(0, pchunk, body, 0)
            oh[p] = [
                pltpu.async_copy(
                    rows_a.at[pl.ds((s * n_batch + bb) * pchunk, pchunk)],
                    out_hbm.at[pl.ds(bb * seq_len + pos0 + roff, pchunk)],
                    osem[s],
                )
                for bb in range(n_batch)
            ]

        for p in range(npc):
            if p >= depth:
                for h in oh[p - depth]:
                    h.wait()
            start(p)
            if p >= 1:
                process(p - 1)
        process(npc - 1)
        for p in range(npc - depth, npc):
            for h in oh[p]:
                h.wait()

    return emb_kernel(ids_flat, word_emb, pos_emb)


def kernel(input_ids, word_emb, pos_emb):
    b, s = input_ids.shape
    _, d = word_emb.shape
    ids_flat = input_ids.reshape(b * s).astype(jnp.int32)
    out = _emb_call(ids_flat, word_emb, pos_emb, b, s)
    return out.reshape(b, s, d)


# speedup vs baseline: 1.0123x; 1.0123x over previous
import functools

import jax
import jax.numpy as jnp
from jax import lax
from jax.experimental import pallas as pl
from jax.experimental.pallas import tpu as pltpu
from jax.experimental.pallas import tpu_sc as plsc

_PCHUNK = 8
_DEPTH = 3


def _emb_call(ids_flat, word_emb, pos_emb, n_batch, seq_len):
    n_rows, = ids_flat.shape
    _, d = word_emb.shape

    info = plsc.get_sparse_core_info()
    nc, ns, lanes = info.num_cores, info.num_subcores, info.num_lanes
    nw = nc * ns
    b_per_w = n_rows // nw
    s_per_w = seq_len // nw
    pchunk = _PCHUNK
    npc = s_per_w // pchunk
    depth = _DEPTH

    mesh = plsc.VectorSubcoreMesh(core_axis_name="c", subcore_axis_name="s")

    scratch = [
        pltpu.VMEM((b_per_w,), jnp.int32),
        pltpu.VMEM((depth * pchunk, d), jnp.float32),
        pltpu.VMEM((depth * n_batch * pchunk, d), jnp.float32),
    ]
    scratch += [pltpu.SemaphoreType.DMA for _ in range(3 * depth + 1)]

    @functools.partial(
        pl.kernel,
        mesh=mesh,
        out_type=jax.ShapeDtypeStruct((n_rows, d), jnp.float32),
        scratch_types=scratch,
    )
    def emb_kernel(ids_hbm, wemb_hbm, pemb_hbm, out_hbm, idx_v, pos_a, rows_a,
                   *sems):
        psem = sems[:depth]
        gsem = sems[depth:2 * depth]
        osem = sems[2 * depth:3 * depth]
        isem = sems[3 * depth]

        wid = lax.axis_index("s") * nc + lax.axis_index("c")
        pos0 = wid * s_per_w

        ih = [
            pltpu.async_copy(
                ids_hbm.at[pl.ds(bb * seq_len + pos0, s_per_w)],
                idx_v.at[pl.ds(bb * s_per_w, s_per_w)],
                isem,
            )
            for bb in range(n_batch)
        ]

        ph = [None] * npc
        gh = [None] * npc
        oh = [None] * npc

        for p in range(min(depth, npc)):
            ph[p] = pltpu.async_copy(
                pemb_hbm.at[pl.ds(pos0 + p * pchunk, pchunk)],
                pos_a.at[pl.ds(p * pchunk, pchunk)],
                psem[p],
            )
        for h in ih:
            h.wait()

        def start(p):
            s = p % depth
            roff = p * pchunk
            if p >= depth:
                ph[p] = pltpu.async_copy(
                    pemb_hbm.at[pl.ds(pos0 + roff, pchunk)],
                    pos_a.at[pl.ds(s * pchunk, pchunk)],
                    psem[s],
                )
            gh[p] = [
                pltpu.async_copy(
                    wemb_hbm.at[idx_v.at[pl.ds(bb * s_per_w + roff, pchunk)]],
                    rows_a.at[pl.ds((s * n_batch + bb) * pchunk, pchunk)],
                    gsem[s],
                )
                for bb in range(n_batch)
            ]

        def process(p):
            s = p % depth
            roff = p * pchunk
            ph[p].wait()
            for h in gh[p]:
                h.wait()

            def body(r, carry):
                for j in range(d // lanes):
                    sl = pl.ds(j * lanes, lanes)
                    pv = pos_a[s * pchunk + r, sl]
                    for bb in range(n_batch):
                        plsc.addupdate(
                            rows_a.at[(s * n_batch + bb) * pchunk + r, sl], pv
                        )
                return carry

            lax.fori_loop(0, pchunk, body, 0)
            oh[p] = [
                pltpu.async_copy(
                    rows_a.at[pl.ds((s * n_batch + bb) * pchunk, pchunk)],
                    out_hbm.at[pl.ds(bb * seq_len + pos0 + roff, pchunk)],
                    osem[s],
                )
                for bb in range(n_batch)
            ]

        for p in range(npc):
            if p >= depth:
                for h in oh[p - depth]:
                    h.wait()
            start(p)
            if p >= 1:
                process(p - 1)
        process(npc - 1)
        for p in range(npc - depth, npc):
            for h in oh[p]:
                h.wait()

    return emb_kernel(ids_flat, word_emb, pos_emb)


def kernel(input_ids, word_emb, pos_emb):
    b, s = input_ids.shape
    _, d = word_emb.shape
    ids_flat = input_ids.reshape(b * s).astype(jnp.int32)
    out = _emb_call(ids_flat, word_emb, pos_emb, b, s)
    return out.reshape(b, s, d)
